# Initial kernel scaffold; baseline (speedup 1.0000x reference)
import jax
import jax.numpy as jnp
from jax.experimental import pallas as pl


def kernel(x, edge_index, edge_attr, W1, b1, W2, b2):
    raise NotImplementedError("write your pallas kernel here")


# speedup vs baseline: 15.7274x; 15.7274x over previous
import functools

import jax
import jax.numpy as jnp
from jax import lax
from jax.experimental import pallas as pl
from jax.experimental.pallas import tpu as pltpu
from jax.experimental.pallas import tpu_sc as plsc

F32 = jnp.float32
I32 = jnp.int32

NC = 2
NS = 16
NW = NC * NS
L = 16

CH = 128


def _sc_mesh():
    return plsc.VectorSubcoreMesh(core_axis_name="c", subcore_axis_name="s")


@functools.lru_cache(maxsize=None)
def _make_deg_kernel(N, E):
    assert E % CH == 0 and N % NS == 0
    NCHUNK = E // CH
    bounds = [(NCHUNK * t) // NW for t in range(NW + 1)]
    MAXC = max(bounds[t + 1] - bounds[t] for t in range(NW))
    RPT = N // NS
    nfull = RPT // CH
    rem = RPT % CH

    @functools.partial(
        pl.kernel,
        mesh=_sc_mesh(),
        out_type=jax.ShapeDtypeStruct((NC * N, L), F32),
        compiler_params=pltpu.CompilerParams(use_tc_tiling_on_sc=False),
        scratch_types=[
            pltpu.VMEM((MAXC, CH), I32),
            pltpu.VMEM((CH, L), F32),
            pltpu.VMEM_SHARED((N, L), F32),
        ],
    )
    def deg_kernel(row_hbm, out_hbm, ridx, ones_v, acc):
        c = lax.axis_index("c")
        s = lax.axis_index("s")
        t = c * NS + s
        zeros = jnp.zeros((L,), F32)
        ones = jnp.ones((L,), F32)

        start = (t * NCHUNK) // NW
        nct = ((t + 1) * NCHUNK) // NW - start
        pltpu.sync_copy(row_hbm.at[pl.ds(start, MAXC)], ridx)

        def zf(i, carry):
            ones_v[i, pl.ds(0, L)] = zeros
            return carry

        lax.fori_loop(0, CH, zf, None)
        for q in range(nfull):
            pltpu.sync_copy(ones_v, acc.at[pl.ds(s * RPT + q * CH, CH)])
        if rem:
            pltpu.sync_copy(ones_v.at[pl.ds(0, rem)],
                            acc.at[pl.ds(s * RPT + nfull * CH, rem)])

        def of(i, carry):
            ones_v[i, pl.ds(0, L)] = ones
            return carry

        lax.fori_loop(0, CH, of, None)
        plsc.subcore_barrier()

        def chunk_body(k, carry):
            pltpu.sync_copy(ones_v, acc.at[ridx.at[k]], add=True)
            return carry

        lax.fori_loop(0, nct, chunk_body, None)
        plsc.subcore_barrier()

        for q in range(nfull):
            pltpu.sync_copy(acc.at[pl.ds(s * RPT + q * CH, CH)], ones_v)
            pltpu.sync_copy(ones_v,
                            out_hbm.at[pl.ds(c * N + s * RPT + q * CH, CH)])
        if rem:
            pltpu.sync_copy(acc.at[pl.ds(s * RPT + nfull * CH, rem)],
                            ones_v.at[pl.ds(0, rem)])
            pltpu.sync_copy(
                ones_v.at[pl.ds(0, rem)],
                out_hbm.at[pl.ds(c * N + s * RPT + nfull * CH, rem)])

    return deg_kernel


@functools.lru_cache(maxsize=None)
def _make_spmm_kernel(N, E, D):
    assert E % CH == 0 and D % L == 0 and N % NS == 0
    NCHUNK = E // CH
    bounds = [(NCHUNK * t) // NW for t in range(NW + 1)]
    MAXC = max(bounds[t + 1] - bounds[t] for t in range(NW))
    RPT = N // NS
    nfull = RPT // CH
    rem = RPT % CH

    @functools.partial(
        pl.kernel,
        mesh=_sc_mesh(),
        out_type=jax.ShapeDtypeStruct((NC * N, D), F32),
        compiler_params=pltpu.CompilerParams(use_tc_tiling_on_sc=False),
        scratch_types=[
            pltpu.VMEM((MAXC, CH), I32),
            pltpu.VMEM((MAXC, CH), I32),
            pltpu.VMEM((MAXC, CH), F32),
            pltpu.VMEM((CH, D), F32),
            pltpu.VMEM_SHARED((N, D), F32),
            pltpu.SemaphoreType.DMA,
        ],
    )
    def spmm_kernel(z_hbm, row_hbm, col_hbm, ew_hbm, out_hbm,
                    ridx, cidx, ewv, rows_v, acc, sem):
        c = lax.axis_index("c")
        s = lax.axis_index("s")
        t = c * NS + s
        zeros = jnp.zeros((L,), F32)

        def zr(i, carry):
            for j in range(D // L):
                rows_v[i, pl.ds(j * L, L)] = zeros
            return carry

        lax.fori_loop(0, CH, zr, None)

        for q in range(nfull):
            pltpu.sync_copy(rows_v, acc.at[pl.ds(s * RPT + q * CH, CH)])
        if rem:
            pltpu.sync_copy(rows_v.at[pl.ds(0, rem)],
                            acc.at[pl.ds(s * RPT + nfull * CH, rem)])
        plsc.subcore_barrier()

        start = (t * NCHUNK) // NW
        nct = ((t + 1) * NCHUNK) // NW - start
        pltpu.sync_copy(row_hbm.at[pl.ds(start, MAXC)], ridx)
        pltpu.sync_copy(col_hbm.at[pl.ds(start, MAXC)], cidx)
        pltpu.sync_copy(ew_hbm.at[pl.ds(start, MAXC)], ewv)

        def chunk_body(k, carry):
            pltpu.async_copy(z_hbm.at[ridx.at[k]], rows_v, sem).wait()

            def scale(g, c2):
                wv = ewv[k, pl.ds(g * L, L)]
                for lidx in range(L):
                    w = wv[lidx]
                    r = g * L + lidx
                    for j in range(D // L):
                        rows_v[r, pl.ds(j * L, L)] = \
                            rows_v[r, pl.ds(j * L, L)] * w
                return c2

            lax.fori_loop(0, CH // L, scale, None)
            pltpu.sync_copy(rows_v, acc.at[cidx.at[k]], add=True)
            return carry

        lax.fori_loop(0, nct, chunk_body, None)
        plsc.subcore_barrier()

        for q in range(nfull):
            pltpu.sync_copy(acc.at[pl.ds(s * RPT + q * CH, CH)], rows_v)
            pltpu.sync_copy(rows_v,
                            out_hbm.at[pl.ds(c * N + s * RPT + q * CH, CH)])
        if rem:
            pltpu.sync_copy(acc.at[pl.ds(s * RPT + nfull * CH, rem)],
                            rows_v.at[pl.ds(0, rem)])
            pltpu.sync_copy(
                rows_v.at[pl.ds(0, rem)],
                out_hbm.at[pl.ds(c * N + s * RPT + nfull * CH, rem)])

    return spmm_kernel


_ROWS = 1000


def _tc_matmul(x, W):
    N, Din = x.shape
    K = W.shape[1]

    def body(x_ref, w_ref, o_ref):
        o_ref[...] = jnp.dot(x_ref[...], w_ref[...],
                             preferred_element_type=F32)

    return pl.pallas_call(
        body,
        grid=(N // _ROWS,),
        in_specs=[pl.BlockSpec((_ROWS, Din), lambda i: (i, 0)),
                  pl.BlockSpec((Din, K), lambda i: (0, 0))],
        out_specs=pl.BlockSpec((_ROWS, K), lambda i: (i, 0)),
        out_shape=jax.ShapeDtypeStruct((N, K), F32),
    )(x, W)


def _tc_scale(degp0, degp1, y1):
    N, D = y1.shape

    def body(d0_ref, d1_ref, y_ref, z_ref, dis_ref):
        d = d0_ref[...] + d1_ref[...] + 1.0
        dis = lax.rsqrt(d)[:, 0:1]
        dis_ref[...] = dis
        z_ref[...] = y_ref[...] * dis

    return pl.pallas_call(
        body,
        grid=(N // _ROWS,),
        in_specs=[pl.BlockSpec((_ROWS, L), lambda i: (i, 0)),
                  pl.BlockSpec((_ROWS, L), lambda i: (i, 0)),
                  pl.BlockSpec((_ROWS, D), lambda i: (i, 0))],
        out_specs=[pl.BlockSpec((_ROWS, D), lambda i: (i, 0)),
                   pl.BlockSpec((_ROWS, 1), lambda i: (i, 0))],
        out_shape=[jax.ShapeDtypeStruct((N, D), F32),
                   jax.ShapeDtypeStruct((N, 1), F32)],
    )(degp0, degp1, y1)


def _tc_layer2(dis, pa, pb, z1, b1, W2):
    N, D = z1.shape
    K = W2.shape[1]

    def body(dis_ref, pa_ref, pb_ref, z1_ref, b1_ref, w_ref, z2_ref):
        dis = dis_ref[...]
        h = (pa_ref[...] + pb_ref[...] + z1_ref[...]) * dis + b1_ref[...]
        h = jnp.maximum(h, 0.0)
        z2_ref[...] = jnp.dot(h, w_ref[...], preferred_element_type=F32) * dis

    return pl.pallas_call(
        body,
        grid=(N // _ROWS,),
        in_specs=[pl.BlockSpec((_ROWS, 1), lambda i: (i, 0)),
                  pl.BlockSpec((_ROWS, D), lambda i: (i, 0)),
                  pl.BlockSpec((_ROWS, D), lambda i: (i, 0)),
                  pl.BlockSpec((_ROWS, D), lambda i: (i, 0)),
                  pl.BlockSpec((1, D), lambda i: (0, 0)),
                  pl.BlockSpec((D, K), lambda i: (0, 0))],
        out_specs=pl.BlockSpec((_ROWS, K), lambda i: (i, 0)),
        out_shape=jax.ShapeDtypeStruct((N, K), F32),
    )(dis, pa, pb, z1, b1, W2)


def _tc_out(dis, pa, pb, z2, b2):
    N, K = z2.shape

    def body(dis_ref, pa_ref, pb_ref, z2_ref, b2_ref, o_ref):
        v = (pa_ref[...] + pb_ref[...] + z2_ref[...]) * dis_ref[...] \
            + b2_ref[...]
        v = v - jnp.max(v, axis=1, keepdims=True)
        o_ref[...] = v - jnp.log(jnp.sum(jnp.exp(v), axis=1, keepdims=True))

    return pl.pallas_call(
        body,
        grid=(N // _ROWS,),
        in_specs=[pl.BlockSpec((_ROWS, 1), lambda i: (i, 0)),
                  pl.BlockSpec((_ROWS, K), lambda i: (i, 0)),
                  pl.BlockSpec((_ROWS, K), lambda i: (i, 0)),
                  pl.BlockSpec((_ROWS, K), lambda i: (i, 0)),
                  pl.BlockSpec((1, K), lambda i: (0, 0))],
        out_specs=pl.BlockSpec((_ROWS, K), lambda i: (i, 0)),
        out_shape=jax.ShapeDtypeStruct((N, K), F32),
    )(dis, pa, pb, z2, b2)


def kernel(x, edge_index, edge_attr, W1, b1, W2, b2):
    N, Din = x.shape
    E = edge_index.shape[1]
    Dh = W1.shape[1]
    Dout = W2.shape[1]

    row = edge_index[0]
    col = edge_index[1]
    row2 = row.reshape(E // CH, CH)
    col2 = col.reshape(E // CH, CH)
    ew2 = edge_attr.reshape(E // CH, CH)

    degp = _make_deg_kernel(N, E)(row2).reshape(NC, N, L)
    y1 = _tc_matmul(x, W1)
    z1, dis = _tc_scale(degp[0], degp[1], y1)

    p1 = _make_spmm_kernel(N, E, Dh)(z1, row2, col2, ew2).reshape(NC, N, Dh)
    z2 = _tc_layer2(dis, p1[0], p1[1], z1, b1.reshape(1, Dh), W2)

    p2 = _make_spmm_kernel(N, E, Dout)(z2, row2, col2, ew2).reshape(NC, N, Dout)
    return _tc_out(dis, p2[0], p2[1], z2, b2.reshape(1, Dout))
